# baseline (speedup 1.0000x reference)
import jax
import jax.numpy as jnp
from jax.experimental import pallas as pl
from jax.experimental.pallas import tpu as pltpu

_BF = jnp.bfloat16


def _dot(a, b, dims=((1,), (0,))):
    return jax.lax.dot_general(a.astype(_BF), b.astype(_BF),
                               (dims, ((), ())),
                               preferred_element_type=jnp.float32)


def _build_p(p_ref):
    n = p_ref.shape[0]
    q = jax.lax.broadcasted_iota(jnp.int32, (n, n), 0)
    r = jax.lax.broadcasted_iota(jnp.int32, (n, n), 1)
    nf = n // 4
    p_ref[...] = (r == 4 * (q % nf) + q // nf).astype(_BF)


def _regroup(t, wcat, p_ref):
    tb = _dot(t, wcat).astype(_BF)
    nf = p_ref.shape[0] // 4
    no = tb.shape[1] // 4
    acc = None
    for ci in range(4):
        part = _dot(p_ref[ci * nf:(ci + 1) * nf, :],
                    tb[:, ci * no:(ci + 1) * no])
        acc = part if acc is None else acc + part
    return acc


def _f_body(di0_ref, di1_ref, vr_ref, f_ref, w1_ref, b1_ref, w2cat_ref,
            b2_ref, y0_ref, y1_ref, p_ref):
    b, m = pl.program_id(0), pl.program_id(1)

    @pl.when((b == 0) & (m == 0))
    def _():
        _build_p(p_ref)

    kh = di0_ref.shape[2]
    t = (_dot(di0_ref[0], vr_ref[0, :kh, :]) +
         _dot(di1_ref[0], vr_ref[0, kh:, :]))
    y1_ref[0] = _regroup(t, w2cat_ref[...], p_ref) + b2_ref[...]
    y0_ref[0] = _dot(f_ref[0], w1_ref[...], ((1,), (1,))) + b1_ref[...]


def _n_body(dia0_ref, dia1_ref, fr_ref, l_ref, v_ref, w3_ref, b3_ref,
            w4cat_ref, b4_ref, w5_ref, b5_ref, y2_ref, y3_ref, y4_ref,
            p_ref, c_ref):
    b, m = pl.program_id(0), pl.program_id(1)

    @pl.when((b == 0) & (m == 0))
    def _():
        _build_p(p_ref)

    @pl.when(m == 0)
    def _():
        c_ref[...] = _dot(v_ref[0], w5_ref[...], ((1,), (1,))).astype(_BF)

    kh = dia0_ref.shape[2]
    t = (_dot(dia0_ref[0], fr_ref[0, :kh, :]) +
         _dot(dia1_ref[0], fr_ref[0, kh:, :]))
    y3_ref[0] = _regroup(t, w4cat_ref[...], p_ref) + b4_ref[...]
    y4_ref[0] = _dot(l_ref[0], c_ref[...]) + b5_ref[...]
    nb = y2_ref.shape[1]
    vblk = v_ref[0, pl.ds(m * nb, nb), :]
    y2_ref[0] = _dot(vblk, w3_ref[...], ((1,), (1,))) + b3_ref[...]


def _stack_w(w):
    no, ni = w.shape
    return w.reshape(no, 4, ni // 4).transpose(2, 1, 0).reshape(ni // 4, 4 * no)


def kernel(L, Di, DiA, v, f, W1, b1, W2, b2, W3, b3, W4, b4, W5, b5):
    bsz, n_nodes, ni = v.shape
    n_faces = f.shape[1]
    no = W1.shape[0]
    nc = ni // 4

    vr = v.reshape(bsz, 4 * n_nodes, nc)
    fr = f.reshape(bsz, 4 * n_faces, nc)
    w2cat = _stack_w(W2)
    w4cat = _stack_w(W4)
    b1r, b2r, b3r, b4r, b5r = (x.reshape(1, no) for x in (b1, b2, b3, b4, b5))
    fixed = lambda b, m: (0, 0)
    bm = 1024
    bf = bm // 4
    bmn = 512
    bfn = bmn // 4

    y0, y1 = pl.pallas_call(
        _f_body,
        grid=(bsz, 4 * n_faces // bm),
        in_specs=[
            pl.BlockSpec((1, bm, 2 * n_nodes), lambda b, m: (b, m, 0)),
            pl.BlockSpec((1, bm, 2 * n_nodes), lambda b, m: (b, m, 1)),
            pl.BlockSpec((1, 4 * n_nodes, nc), lambda b, m: (b, 0, 0)),
            pl.BlockSpec((1, bf, ni), lambda b, m: (b, m, 0)),
            pl.BlockSpec((no, ni), fixed),
            pl.BlockSpec((1, no), fixed),
            pl.BlockSpec((nc, 4 * no), fixed),
            pl.BlockSpec((1, no), fixed),
        ],
        out_specs=[
            pl.BlockSpec((1, bf, no), lambda b, m: (b, m, 0)),
            pl.BlockSpec((1, bf, no), lambda b, m: (b, m, 0)),
        ],
        out_shape=[
            jax.ShapeDtypeStruct((bsz, n_faces, no), jnp.float32),
            jax.ShapeDtypeStruct((bsz, n_faces, no), jnp.float32),
        ],
        scratch_shapes=[pltpu.VMEM((bm, bm), _BF)],
    )(Di, Di, vr, f, W1, b1r, w2cat, b2r)

    y2, y3, y4 = pl.pallas_call(
        _n_body,
        grid=(bsz, 4 * n_nodes // bmn),
        in_specs=[
            pl.BlockSpec((1, bmn, 2 * n_faces), lambda b, m: (b, m, 0)),
            pl.BlockSpec((1, bmn, 2 * n_faces), lambda b, m: (b, m, 1)),
            pl.BlockSpec((1, 4 * n_faces, nc), lambda b, m: (b, 0, 0)),
            pl.BlockSpec((1, bfn, n_nodes), lambda b, m: (b, m, 0)),
            pl.BlockSpec((1, n_nodes, ni), lambda b, m: (b, 0, 0)),
            pl.BlockSpec((no, ni), fixed),
            pl.BlockSpec((1, no), fixed),
            pl.BlockSpec((nc, 4 * no), fixed),
            pl.BlockSpec((1, no), fixed),
            pl.BlockSpec((no, ni), fixed),
            pl.BlockSpec((1, no), fixed),
        ],
        out_specs=[
            pl.BlockSpec((1, bfn, no), lambda b, m: (b, m, 0)),
            pl.BlockSpec((1, bfn, no), lambda b, m: (b, m, 0)),
            pl.BlockSpec((1, bfn, no), lambda b, m: (b, m, 0)),
        ],
        out_shape=[
            jax.ShapeDtypeStruct((bsz, n_nodes, no), jnp.float32),
            jax.ShapeDtypeStruct((bsz, n_nodes, no), jnp.float32),
            jax.ShapeDtypeStruct((bsz, n_nodes, no), jnp.float32),
        ],
        scratch_shapes=[pltpu.VMEM((bmn, bmn), _BF),
                        pltpu.VMEM((n_nodes, no), _BF)],
    )(DiA, DiA, fr, L, v, W3, b3r, w4cat, b4r, W5, b5r)

    return (y0, y1, y2, y3, y4)


# speedup vs baseline: 1.0277x; 1.0277x over previous
import jax
import jax.numpy as jnp
from jax.experimental import pallas as pl
from jax.experimental.pallas import tpu as pltpu

_BF = jnp.bfloat16


def _dot(a, b, dims=((1,), (0,))):
    return jax.lax.dot_general(a.astype(_BF), b.astype(_BF),
                               (dims, ((), ())),
                               preferred_element_type=jnp.float32)


def _build_p(p_ref):
    n = p_ref.shape[0]
    q = jax.lax.broadcasted_iota(jnp.int32, (n, n), 0)
    r = jax.lax.broadcasted_iota(jnp.int32, (n, n), 1)
    nf = n // 4
    p_ref[...] = (r == 4 * (q % nf) + q // nf).astype(_BF)


def _regroup(t, wcat, p_ref):
    tb = _dot(t, wcat).astype(_BF)
    nf = p_ref.shape[0] // 4
    no = tb.shape[1] // 4
    acc = None
    for ci in range(4):
        part = _dot(p_ref[ci * nf:(ci + 1) * nf, :],
                    tb[:, ci * no:(ci + 1) * no])
        acc = part if acc is None else acc + part
    return acc


def _f_body(di_ref, vr_ref, f_ref, w1_ref, b1_ref, w2cat_ref,
            b2_ref, y0_ref, y1_ref, p_ref):
    b, m = pl.program_id(0), pl.program_id(1)

    @pl.when((b == 0) & (m == 0))
    def _():
        _build_p(p_ref)

    t = _dot(di_ref[0], vr_ref[0])
    y1_ref[0] = _regroup(t, w2cat_ref[...], p_ref) + b2_ref[...]
    y0_ref[0] = _dot(f_ref[0], w1_ref[...], ((1,), (1,))) + b1_ref[...]


def _n_body(dia_ref, fr_ref, l_ref, v_ref, w3_ref, b3_ref,
            w4cat_ref, b4_ref, w5_ref, b5_ref, y2_ref, y3_ref, y4_ref,
            p_ref, c_ref):
    b, m = pl.program_id(0), pl.program_id(1)

    @pl.when((b == 0) & (m == 0))
    def _():
        _build_p(p_ref)

    @pl.when(m == 0)
    def _():
        c_ref[...] = _dot(v_ref[0], w5_ref[...], ((1,), (1,))).astype(_BF)

    t = _dot(dia_ref[0], fr_ref[0])
    y3_ref[0] = _regroup(t, w4cat_ref[...], p_ref) + b4_ref[...]
    y4_ref[0] = _dot(l_ref[0], c_ref[...]) + b5_ref[...]
    nb = y2_ref.shape[1]
    vblk = v_ref[0, pl.ds(m * nb, nb), :]
    y2_ref[0] = _dot(vblk, w3_ref[...], ((1,), (1,))) + b3_ref[...]


def _stack_w(w):
    no, ni = w.shape
    return w.reshape(no, 4, ni // 4).transpose(2, 1, 0).reshape(ni // 4, 4 * no)


def kernel(L, Di, DiA, v, f, W1, b1, W2, b2, W3, b3, W4, b4, W5, b5):
    bsz, n_nodes, ni = v.shape
    n_faces = f.shape[1]
    no = W1.shape[0]
    nc = ni // 4

    vr = v.reshape(bsz, 4 * n_nodes, nc)
    fr = f.reshape(bsz, 4 * n_faces, nc)
    w2cat = _stack_w(W2)
    w4cat = _stack_w(W4)
    b1r, b2r, b3r, b4r, b5r = (x.reshape(1, no) for x in (b1, b2, b3, b4, b5))
    fixed = lambda b, m: (0, 0)
    bm = 1024
    bf = bm // 4
    bmn = 512
    bfn = bmn // 4

    y0, y1 = pl.pallas_call(
        _f_body,
        grid=(bsz, 4 * n_faces // bm),
        in_specs=[
            pl.BlockSpec((1, bm, 4 * n_nodes), lambda b, m: (b, m, 0)),
            pl.BlockSpec((1, 4 * n_nodes, nc), lambda b, m: (b, 0, 0)),
            pl.BlockSpec((1, bf, ni), lambda b, m: (b, m, 0)),
            pl.BlockSpec((no, ni), fixed),
            pl.BlockSpec((1, no), fixed),
            pl.BlockSpec((nc, 4 * no), fixed),
            pl.BlockSpec((1, no), fixed),
        ],
        out_specs=[
            pl.BlockSpec((1, bf, no), lambda b, m: (b, m, 0)),
            pl.BlockSpec((1, bf, no), lambda b, m: (b, m, 0)),
        ],
        out_shape=[
            jax.ShapeDtypeStruct((bsz, n_faces, no), jnp.float32),
            jax.ShapeDtypeStruct((bsz, n_faces, no), jnp.float32),
        ],
        scratch_shapes=[pltpu.VMEM((bm, bm), _BF)],
    )(Di, vr, f, W1, b1r, w2cat, b2r)

    y2, y3, y4 = pl.pallas_call(
        _n_body,
        grid=(bsz, 4 * n_nodes // bmn),
        in_specs=[
            pl.BlockSpec((1, bmn, 4 * n_faces), lambda b, m: (b, m, 0)),
            pl.BlockSpec((1, 4 * n_faces, nc), lambda b, m: (b, 0, 0)),
            pl.BlockSpec((1, bfn, n_nodes), lambda b, m: (b, m, 0)),
            pl.BlockSpec((1, n_nodes, ni), lambda b, m: (b, 0, 0)),
            pl.BlockSpec((no, ni), fixed),
            pl.BlockSpec((1, no), fixed),
            pl.BlockSpec((nc, 4 * no), fixed),
            pl.BlockSpec((1, no), fixed),
            pl.BlockSpec((no, ni), fixed),
            pl.BlockSpec((1, no), fixed),
        ],
        out_specs=[
            pl.BlockSpec((1, bfn, no), lambda b, m: (b, m, 0)),
            pl.BlockSpec((1, bfn, no), lambda b, m: (b, m, 0)),
            pl.BlockSpec((1, bfn, no), lambda b, m: (b, m, 0)),
        ],
        out_shape=[
            jax.ShapeDtypeStruct((bsz, n_nodes, no), jnp.float32),
            jax.ShapeDtypeStruct((bsz, n_nodes, no), jnp.float32),
            jax.ShapeDtypeStruct((bsz, n_nodes, no), jnp.float32),
        ],
        scratch_shapes=[pltpu.VMEM((bmn, bmn), _BF),
                        pltpu.VMEM((n_nodes, no), _BF)],
    )(DiA, fr, L, v, W3, b3r, w4cat, b4r, W5, b5r)

    return (y0, y1, y2, y3, y4)


# speedup vs baseline: 1.0716x; 1.0427x over previous
import jax
import jax.numpy as jnp
from jax.experimental import pallas as pl
from jax.experimental.pallas import tpu as pltpu

_BF = jnp.bfloat16


def _dot(a, b, dims=((1,), (0,))):
    return jax.lax.dot_general(a.astype(_BF), b.astype(_BF),
                               (dims, ((), ())),
                               preferred_element_type=jnp.float32)


def _regroup(t, w, p_ref, nf):
    tb = t.astype(_BF)
    nq = p_ref.shape[0] // 4
    kk = t.shape[0]
    acc = None
    for ci in range(4):
        sel = _dot(p_ref[ci * nq:ci * nq + nf, :kk], tb)
        part = _dot(sel, w[:, 32 * ci:32 * (ci + 1)], ((1,), (1,)))
        acc = part if acc is None else acc + part
    return acc


def _body(di_ref, dia_ref, vr_ref, fr_ref, f_ref, v_ref, l_ref,
          w1_ref, b1_ref, w2_ref, b2_ref, w3_ref, b3_ref, w4_ref, b4_ref,
          w5_ref, b5_ref, y0_ref, y1_ref, y2_ref, y3_ref, y4_ref,
          p_ref, c_ref):
    b, m = pl.program_id(0), pl.program_id(1)

    @pl.when((b == 0) & (m == 0))
    def _():
        n = p_ref.shape[0]
        q = jax.lax.broadcasted_iota(jnp.int32, (n, n), 0)
        r = jax.lax.broadcasted_iota(jnp.int32, (n, n), 1)
        nq = n // 4
        p_ref[...] = (r == 4 * (q % nq) + q // nq).astype(_BF)

    @pl.when(m == 0)
    def _():
        c_ref[...] = _dot(v_ref[0], w5_ref[...], ((1,), (1,))).astype(_BF)

    t = _dot(di_ref[0], vr_ref[0])
    y1_ref[0] = _regroup(t, w2_ref[...], p_ref, y1_ref.shape[1]) + b2_ref[...]
    y0_ref[0] = _dot(f_ref[0], w1_ref[...], ((1,), (1,))) + b1_ref[...]

    tn = _dot(dia_ref[0], fr_ref[0])
    y3_ref[0] = _regroup(tn, w4_ref[...], p_ref, y3_ref.shape[1]) + b4_ref[...]
    y4_ref[0] = _dot(l_ref[0], c_ref[...]) + b5_ref[...]
    nb = y2_ref.shape[1]
    vblk = v_ref[0, pl.ds(m * nb, nb), :]
    y2_ref[0] = _dot(vblk, w3_ref[...], ((1,), (1,))) + b3_ref[...]


def kernel(L, Di, DiA, v, f, W1, b1, W2, b2, W3, b3, W4, b4, W5, b5):
    bsz, n_nodes, ni = v.shape
    n_faces = f.shape[1]
    no = W1.shape[0]
    nc = ni // 4

    vr = v.reshape(bsz, 4 * n_nodes, nc)
    fr = f.reshape(bsz, 4 * n_faces, nc)
    b1r, b2r, b3r, b4r, b5r = (x.reshape(1, no) for x in (b1, b2, b3, b4, b5))
    fixed = lambda b, m: (0, 0)
    ng = 4
    bmf = 4 * n_faces // ng
    bff = bmf // 4
    bmn = 4 * n_nodes // ng
    bfn = bmn // 4

    outs = pl.pallas_call(
        _body,
        grid=(bsz, ng),
        in_specs=[
            pl.BlockSpec((1, bmf, 4 * n_nodes), lambda b, m: (b, m, 0)),
            pl.BlockSpec((1, bmn, 4 * n_faces), lambda b, m: (b, m, 0)),
            pl.BlockSpec((1, 4 * n_nodes, nc), lambda b, m: (b, 0, 0)),
            pl.BlockSpec((1, 4 * n_faces, nc), lambda b, m: (b, 0, 0)),
            pl.BlockSpec((1, bff, ni), lambda b, m: (b, m, 0)),
            pl.BlockSpec((1, n_nodes, ni), lambda b, m: (b, 0, 0)),
            pl.BlockSpec((1, bfn, n_nodes), lambda b, m: (b, m, 0)),
            pl.BlockSpec((no, ni), fixed),
            pl.BlockSpec((1, no), fixed),
            pl.BlockSpec((no, ni), fixed),
            pl.BlockSpec((1, no), fixed),
            pl.BlockSpec((no, ni), fixed),
            pl.BlockSpec((1, no), fixed),
            pl.BlockSpec((no, ni), fixed),
            pl.BlockSpec((1, no), fixed),
            pl.BlockSpec((no, ni), fixed),
            pl.BlockSpec((1, no), fixed),
        ],
        out_specs=[
            pl.BlockSpec((1, bff, no), lambda b, m: (b, m, 0)),
            pl.BlockSpec((1, bff, no), lambda b, m: (b, m, 0)),
            pl.BlockSpec((1, bfn, no), lambda b, m: (b, m, 0)),
            pl.BlockSpec((1, bfn, no), lambda b, m: (b, m, 0)),
            pl.BlockSpec((1, bfn, no), lambda b, m: (b, m, 0)),
        ],
        out_shape=[
            jax.ShapeDtypeStruct((bsz, n_faces, no), jnp.float32),
            jax.ShapeDtypeStruct((bsz, n_faces, no), jnp.float32),
            jax.ShapeDtypeStruct((bsz, n_nodes, no), jnp.float32),
            jax.ShapeDtypeStruct((bsz, n_nodes, no), jnp.float32),
            jax.ShapeDtypeStruct((bsz, n_nodes, no), jnp.float32),
        ],
        scratch_shapes=[pltpu.VMEM((bmf, bmf), _BF),
                        pltpu.VMEM((n_nodes, no), _BF)],
    )(Di, DiA, vr, fr, f, v, L,
      W1, b1r, W2, b2r, W3, b3r, W4, b4r, W5, b5r)

    y0, y1, y2, y3, y4 = outs
    return (y0, y1, y2, y3, y4)
